# baseline (speedup 1.0000x reference)
import functools

import jax
import jax.numpy as jnp
from jax import lax
from jax.experimental import pallas as pl
from jax.experimental.pallas import tpu as pltpu
from jax.experimental.pallas import tpu_sc as plsc

N = 16 * 512 * 512
NC, NS, L = 2, 16, 16
NW = NC * NS
ROWS_PER_TILE = 256
CROWS = 8
CHUNK = CROWS * 512
NCHUNK = ROWS_PER_TILE // CROWS
SHIFT = 16
NB = 1 << 15
HSIZE = 2 * NB


def _sc_hist_body(pred_hbm, tgt_hbm, hist_hbm, psum_hbm, pred_v, tgt_v,
                  hist_v, acc_v, sem0, sem1):
    cid = lax.axis_index("c")
    sid = lax.axis_index("s")
    wid = sid * NC + cid
    img = wid // 2
    half = wid % 2
    sems = (sem0, sem1)

    zeros16 = jnp.zeros((L,), jnp.float32)
    ones16 = jnp.ones((L,), jnp.float32)

    @plsc.parallel_loop(0, HSIZE // (4 * L), unroll=8)
    def _(i):
        r = i // 2
        c = (i % 2) * 64
        for u in range(4):
            hist_v[r, pl.ds(c + u * L, L)] = zeros16

    def start(k, b):
        r0 = half * ROWS_PER_TILE + k * CROWS
        pltpu.async_copy(pred_hbm.at[img, pl.ds(r0, CROWS)], pred_v.at[b],
                         sems[b])
        pltpu.async_copy(tgt_hbm.at[img, pl.ds(r0, CROWS)], tgt_v.at[b],
                         sems[b])

    def wait(b):
        pltpu.make_async_copy(pred_hbm.at[0, pl.ds(0, CROWS)], pred_v.at[b],
                              sems[b]).wait()
        pltpu.make_async_copy(tgt_hbm.at[0, pl.ds(0, CROWS)], tgt_v.at[b],
                              sems[b]).wait()

    start(0, 0)

    def chunk_pair(g, acc):
        for b in range(2):
            k = 2 * g + b

            @pl.when(k + 1 < NCHUNK)
            def _():
                start(k + 1, 1 - b)

            wait(b)

            def inner(i, acc):
                r = lax.shift_right_logical(i, 5)
                c = (i & 31) * L
                pv = pred_v[b, r, pl.ds(c, L)]
                tv = tgt_v[b, r, pl.ds(c, L)]
                tf = tv.astype(jnp.float32)
                e = 1.0 - pv * (2.0 * tf - 1.0)
                acc = acc + tf
                mask = e > 0.0
                bits = lax.bitcast_convert_type(e, jnp.int32)
                idx = lax.shift_right_logical(bits, SHIFT) + tv * NB
                ridx = lax.shift_right_logical(idx, 7)
                cidx = idx & 127
                ridx = jnp.where(mask, ridx, 0)
                plsc.addupdate_scatter(hist_v, (ridx, cidx), ones16,
                                       mask=mask)
                return acc

            acc = plsc.parallel_loop(0, CHUNK // L, unroll=8,
                                     carry=acc)(inner)
        return acc

    acc = lax.fori_loop(0, NCHUNK // 2, chunk_pair, zeros16)
    acc_v[...] = acc
    pltpu.sync_copy(hist_v, hist_hbm.at[wid])
    pltpu.sync_copy(acc_v, psum_hbm.at[wid])


@functools.cache
def _get_sc_hist():
    return pl.kernel(
        _sc_hist_body,
        out_type=(
            jax.ShapeDtypeStruct((NW, 512, 128), jnp.float32),
            jax.ShapeDtypeStruct((NW, L), jnp.float32),
        ),
        mesh=plsc.VectorSubcoreMesh(core_axis_name="c", subcore_axis_name="s"),
        compiler_params=pltpu.CompilerParams(needs_layout_passes=False),
        scratch_types=[
            pltpu.VMEM((2, CROWS, 512), jnp.float32),
            pltpu.VMEM((2, CROWS, 512), jnp.int32),
            pltpu.VMEM((512, 128), jnp.float32),
            pltpu.VMEM((L,), jnp.float32),
            pltpu.SemaphoreType.DMA,
            pltpu.SemaphoreType.DMA,
        ],
    )


def _suffix_counts(a):
    rows, cols = a.shape
    ii = lax.broadcasted_iota(jnp.int32, (cols, cols), 0)
    jj = lax.broadcasted_iota(jnp.int32, (cols, cols), 1)
    upper = jnp.where(ii <= jj, 1.0, 0.0).astype(jnp.float32)
    pref = jnp.dot(a, upper, preferred_element_type=jnp.float32,
                   precision=lax.Precision.HIGHEST)
    rtot = pref[:, cols - 1:cols]
    ri = lax.broadcasted_iota(jnp.int32, (rows, rows), 0)
    rj = lax.broadcasted_iota(jnp.int32, (rows, rows), 1)
    lstrict = jnp.where(rj < ri, 1.0, 0.0).astype(jnp.float32)
    roff = jnp.dot(lstrict, rtot, preferred_element_type=jnp.float32,
                   precision=lax.Precision.HIGHEST)
    prefix_incl = pref + roff
    total = jnp.sum(a)
    return total - prefix_incl + a


def _tc_reduce_kernel(hist_ref, psum_ref, out_ref):
    h = jnp.sum(hist_ref[...], axis=0)
    hneg = h[:256, :]
    hpos = h[256:, :]
    c = _suffix_counts(hpos)
    f = _suffix_counts(hneg)
    p = jnp.sum(psum_ref[...])

    b = (lax.broadcasted_iota(jnp.int32, (256, 128), 0) * 128
         + lax.broadcasted_iota(jnp.int32, (256, 128), 1))
    v_next = lax.bitcast_convert_type(
        jnp.minimum(b + 1, NB - 1) << SHIFT, jnp.float32)
    v_prev = lax.bitcast_convert_type(
        jnp.maximum(b - 1, 0) << SHIFT, jnp.float32)
    wsum = jnp.where(b >= 32639, 0.0, (v_next - v_prev) * 0.5)

    jac = (c + f) / jnp.maximum(p + f, 1.0)
    out_ref[0, 0] = jnp.sum(jac * wsum)


def kernel(predictions, targets):
    hist, psum = _get_sc_hist()(predictions, targets)
    loss = pl.pallas_call(
        _tc_reduce_kernel,
        out_shape=jax.ShapeDtypeStruct((1, 1), jnp.float32),
        in_specs=[
            pl.BlockSpec(memory_space=pltpu.VMEM),
            pl.BlockSpec(memory_space=pltpu.VMEM),
        ],
        out_specs=pl.BlockSpec(memory_space=pltpu.SMEM),
    )(hist, psum)
    return loss.reshape(())


# speedup vs baseline: 1.0661x; 1.0661x over previous
import functools

import jax
import jax.numpy as jnp
from jax import lax
from jax.experimental import pallas as pl
from jax.experimental.pallas import tpu as pltpu
from jax.experimental.pallas import tpu_sc as plsc

N = 16 * 512 * 512
NC, NS, L = 2, 16, 16
NW = NC * NS
ROWS_PER_TILE = 256
CROWS = 16
CHUNK = CROWS * 512
NCHUNK = ROWS_PER_TILE // CROWS
SHIFT = 16
NB = 1 << 15
HSIZE = 2 * NB


def _sc_hist_body(pred_hbm, tgt_hbm, hist_hbm, psum_hbm, pred_v, tgt_v,
                  hist_v, acc_v, sem0, sem1):
    cid = lax.axis_index("c")
    sid = lax.axis_index("s")
    wid = sid * NC + cid
    img = wid // 2
    half = wid % 2
    sems = (sem0, sem1)

    zeros16 = jnp.zeros((L,), jnp.float32)
    ones16 = jnp.ones((L,), jnp.float32)

    @plsc.parallel_loop(0, HSIZE // (4 * L), unroll=8)
    def _(i):
        r = i // 2
        c = (i % 2) * 64
        for u in range(4):
            hist_v[r, pl.ds(c + u * L, L)] = zeros16

    def start(k, b):
        r0 = half * ROWS_PER_TILE + k * CROWS
        pltpu.async_copy(pred_hbm.at[img, pl.ds(r0, CROWS)], pred_v.at[b],
                         sems[b])
        pltpu.async_copy(tgt_hbm.at[img, pl.ds(r0, CROWS)], tgt_v.at[b],
                         sems[b])

    def wait(b):
        pltpu.make_async_copy(pred_hbm.at[0, pl.ds(0, CROWS)], pred_v.at[b],
                              sems[b]).wait()
        pltpu.make_async_copy(tgt_hbm.at[0, pl.ds(0, CROWS)], tgt_v.at[b],
                              sems[b]).wait()

    start(0, 0)

    def chunk_pair(g, acc):
        for b in range(2):
            k = 2 * g + b

            @pl.when(k + 1 < NCHUNK)
            def _():
                start(k + 1, 1 - b)

            wait(b)

            def inner(i, acc):
                r = lax.shift_right_logical(i, 5)
                c = (i & 31) * L
                pv = pred_v[b, r, pl.ds(c, L)]
                tv = tgt_v[b, r, pl.ds(c, L)]
                tf = tv.astype(jnp.float32)
                e = 1.0 - pv * (2.0 * tf - 1.0)
                acc = acc + tf
                mask = e > 0.0
                bits = lax.bitcast_convert_type(e, jnp.int32)
                idx = lax.shift_right_logical(bits, SHIFT) + tv * NB
                ridx = lax.shift_right_logical(idx, 7)
                cidx = idx & 127
                ridx = jnp.where(mask, ridx, 0)
                plsc.addupdate_scatter(hist_v, (ridx, cidx), ones16,
                                       mask=mask)
                return acc

            acc = plsc.parallel_loop(0, CHUNK // L, unroll=8,
                                     carry=acc)(inner)
        return acc

    acc = lax.fori_loop(0, NCHUNK // 2, chunk_pair, zeros16)
    acc_v[...] = acc
    pltpu.sync_copy(hist_v, hist_hbm.at[wid])
    pltpu.sync_copy(acc_v, psum_hbm.at[wid])


@functools.cache
def _get_sc_hist():
    return pl.kernel(
        _sc_hist_body,
        out_type=(
            jax.ShapeDtypeStruct((NW, 512, 128), jnp.float32),
            jax.ShapeDtypeStruct((NW, L), jnp.float32),
        ),
        mesh=plsc.VectorSubcoreMesh(core_axis_name="c", subcore_axis_name="s"),
        compiler_params=pltpu.CompilerParams(needs_layout_passes=False),
        scratch_types=[
            pltpu.VMEM((2, CROWS, 512), jnp.float32),
            pltpu.VMEM((2, CROWS, 512), jnp.int32),
            pltpu.VMEM((512, 128), jnp.float32),
            pltpu.VMEM((L,), jnp.float32),
            pltpu.SemaphoreType.DMA,
            pltpu.SemaphoreType.DMA,
        ],
    )


def _suffix_counts(a):
    rows, cols = a.shape
    ii = lax.broadcasted_iota(jnp.int32, (cols, cols), 0)
    jj = lax.broadcasted_iota(jnp.int32, (cols, cols), 1)
    upper = jnp.where(ii <= jj, 1.0, 0.0).astype(jnp.float32)
    pref = jnp.dot(a, upper, preferred_element_type=jnp.float32,
                   precision=lax.Precision.HIGHEST)
    rtot = pref[:, cols - 1:cols]
    ri = lax.broadcasted_iota(jnp.int32, (rows, rows), 0)
    rj = lax.broadcasted_iota(jnp.int32, (rows, rows), 1)
    lstrict = jnp.where(rj < ri, 1.0, 0.0).astype(jnp.float32)
    roff = jnp.dot(lstrict, rtot, preferred_element_type=jnp.float32,
                   precision=lax.Precision.HIGHEST)
    prefix_incl = pref + roff
    total = jnp.sum(a)
    return total - prefix_incl + a


def _tc_reduce_kernel(hist_ref, psum_ref, out_ref):
    h = jnp.sum(hist_ref[...], axis=0)
    hneg = h[:256, :]
    hpos = h[256:, :]
    c = _suffix_counts(hpos)
    f = _suffix_counts(hneg)
    p = jnp.sum(psum_ref[...])

    b = (lax.broadcasted_iota(jnp.int32, (256, 128), 0) * 128
         + lax.broadcasted_iota(jnp.int32, (256, 128), 1))
    v_next = lax.bitcast_convert_type(
        jnp.minimum(b + 1, NB - 1) << SHIFT, jnp.float32)
    v_prev = lax.bitcast_convert_type(
        jnp.maximum(b - 1, 0) << SHIFT, jnp.float32)
    wsum = jnp.where(b >= 32639, 0.0, (v_next - v_prev) * 0.5)

    jac = (c + f) / jnp.maximum(p + f, 1.0)
    out_ref[0, 0] = jnp.sum(jac * wsum)


def kernel(predictions, targets):
    hist, psum = _get_sc_hist()(predictions, targets)
    loss = pl.pallas_call(
        _tc_reduce_kernel,
        out_shape=jax.ShapeDtypeStruct((1, 1), jnp.float32),
        in_specs=[
            pl.BlockSpec(memory_space=pltpu.VMEM),
            pl.BlockSpec(memory_space=pltpu.VMEM),
        ],
        out_specs=pl.BlockSpec(memory_space=pltpu.SMEM),
    )(hist, psum)
    return loss.reshape(())


# speedup vs baseline: 1.1069x; 1.0382x over previous
import functools

import jax
import jax.numpy as jnp
from jax import lax
from jax.experimental import pallas as pl
from jax.experimental.pallas import tpu as pltpu
from jax.experimental.pallas import tpu_sc as plsc

N = 16 * 512 * 512
NC, NS, L = 2, 16, 16
NW = NC * NS
ROWS_PER_TILE = 256
CROWS = 16
CHUNK = CROWS * 512
NCHUNK = ROWS_PER_TILE // CROWS
SHIFT = 16
NB = 1 << 15
HSIZE = 2 * NB


def _sc_hist_body(pred_hbm, tgt_hbm, hist_hbm, psum_hbm, pred_v, tgt_v,
                  hist_v, acc_v, sem0, sem1):
    cid = lax.axis_index("c")
    sid = lax.axis_index("s")
    wid = sid * NC + cid
    img = wid // 2
    half = wid % 2
    sems = (sem0, sem1)

    zeros16 = jnp.zeros((L,), jnp.float32)
    ones16 = jnp.ones((L,), jnp.float32)

    @plsc.parallel_loop(0, HSIZE // (4 * L), unroll=8)
    def _(i):
        r = i // 2
        c = (i % 2) * 64
        for u in range(4):
            hist_v[r, pl.ds(c + u * L, L)] = zeros16

    def start(k, b):
        r0 = half * ROWS_PER_TILE + k * CROWS
        pltpu.async_copy(pred_hbm.at[img, pl.ds(r0, CROWS)], pred_v.at[b],
                         sems[b])
        pltpu.async_copy(tgt_hbm.at[img, pl.ds(r0, CROWS)], tgt_v.at[b],
                         sems[b])

    def wait(b):
        pltpu.make_async_copy(pred_hbm.at[0, pl.ds(0, CROWS)], pred_v.at[b],
                              sems[b]).wait()
        pltpu.make_async_copy(tgt_hbm.at[0, pl.ds(0, CROWS)], tgt_v.at[b],
                              sems[b]).wait()

    start(0, 0)

    def chunk_pair(g, acc):
        for b in range(2):
            k = 2 * g + b

            @pl.when(k + 1 < NCHUNK)
            def _():
                start(k + 1, 1 - b)

            wait(b)

            def inner(i, acc):
                r = lax.shift_right_logical(i, 5)
                c = (i & 31) * L
                pv = pred_v[b, r, pl.ds(c, L)]
                tv = tgt_v[b, r, pl.ds(c, L)]
                tf = tv.astype(jnp.float32)
                e = 1.0 - pv * (2.0 * tf - 1.0)
                acc = acc + tf
                mask = e > 0.0
                bits = lax.bitcast_convert_type(e, jnp.int32)
                idx = lax.shift_right_logical(bits, SHIFT) + tv * NB
                ridx = lax.shift_right_logical(idx, 7)
                cidx = idx & 127
                plsc.addupdate_scatter(hist_v, (ridx, cidx), ones16,
                                       mask=mask)
                return acc

            acc = plsc.parallel_loop(0, CHUNK // L, unroll=8,
                                     carry=acc)(inner)
        return acc

    acc = lax.fori_loop(0, NCHUNK // 2, chunk_pair, zeros16)
    acc_v[...] = acc
    pltpu.sync_copy(hist_v, hist_hbm.at[wid])
    pltpu.sync_copy(acc_v, psum_hbm.at[wid])


@functools.cache
def _get_sc_hist():
    return pl.kernel(
        _sc_hist_body,
        out_type=(
            jax.ShapeDtypeStruct((NW, 512, 128), jnp.float32),
            jax.ShapeDtypeStruct((NW, L), jnp.float32),
        ),
        mesh=plsc.VectorSubcoreMesh(core_axis_name="c", subcore_axis_name="s"),
        compiler_params=pltpu.CompilerParams(needs_layout_passes=False),
        scratch_types=[
            pltpu.VMEM((2, CROWS, 512), jnp.float32),
            pltpu.VMEM((2, CROWS, 512), jnp.int32),
            pltpu.VMEM((512, 128), jnp.float32),
            pltpu.VMEM((L,), jnp.float32),
            pltpu.SemaphoreType.DMA,
            pltpu.SemaphoreType.DMA,
        ],
    )


def _suffix_counts(a):
    rows, cols = a.shape
    ii = lax.broadcasted_iota(jnp.int32, (cols, cols), 0)
    jj = lax.broadcasted_iota(jnp.int32, (cols, cols), 1)
    upper = jnp.where(ii <= jj, 1.0, 0.0).astype(jnp.float32)
    pref = jnp.dot(a, upper, preferred_element_type=jnp.float32,
                   precision=lax.Precision.HIGHEST)
    rtot = pref[:, cols - 1:cols]
    ri = lax.broadcasted_iota(jnp.int32, (rows, rows), 0)
    rj = lax.broadcasted_iota(jnp.int32, (rows, rows), 1)
    lstrict = jnp.where(rj < ri, 1.0, 0.0).astype(jnp.float32)
    roff = jnp.dot(lstrict, rtot, preferred_element_type=jnp.float32,
                   precision=lax.Precision.HIGHEST)
    prefix_incl = pref + roff
    total = jnp.sum(a)
    return total - prefix_incl + a


def _tc_reduce_kernel(hist_ref, psum_ref, out_ref):
    h = jnp.sum(hist_ref[...], axis=0)
    hneg = h[:256, :]
    hpos = h[256:, :]
    c = _suffix_counts(hpos)
    f = _suffix_counts(hneg)
    p = jnp.sum(psum_ref[...])

    b = (lax.broadcasted_iota(jnp.int32, (256, 128), 0) * 128
         + lax.broadcasted_iota(jnp.int32, (256, 128), 1))
    v_next = lax.bitcast_convert_type(
        jnp.minimum(b + 1, NB - 1) << SHIFT, jnp.float32)
    v_prev = lax.bitcast_convert_type(
        jnp.maximum(b - 1, 0) << SHIFT, jnp.float32)
    wsum = jnp.where(b >= 32639, 0.0, (v_next - v_prev) * 0.5)

    jac = (c + f) / jnp.maximum(p + f, 1.0)
    out_ref[0, 0] = jnp.sum(jac * wsum)


def kernel(predictions, targets):
    hist, psum = _get_sc_hist()(predictions, targets)
    loss = pl.pallas_call(
        _tc_reduce_kernel,
        out_shape=jax.ShapeDtypeStruct((1, 1), jnp.float32),
        in_specs=[
            pl.BlockSpec(memory_space=pltpu.VMEM),
            pl.BlockSpec(memory_space=pltpu.VMEM),
        ],
        out_specs=pl.BlockSpec(memory_space=pltpu.SMEM),
    )(hist, psum)
    return loss.reshape(())
